# Initial kernel scaffold; baseline (speedup 1.0000x reference)
import functools

import jax
import jax.numpy as jnp
from jax import lax
from jax.experimental import pallas as pl
from jax.experimental.pallas import tpu as pltpu
from jax.experimental.pallas import tpu_sc as plsc

N = 10000
NP = 10240
E = 160000
D = 256
D2 = 128
NCLS = 128
K = 3

NC = 2
NS = 16

ET = E // NS
CH = 80
NCH = ET // CH
PT = NP // NS
RC = 128
NRC = PT // RC

ED = E // (NC * NS)
CHD = 40
NCHD = ED // CHD

_mesh = functools.partial(
    plsc.VectorSubcoreMesh, core_axis_name="c", subcore_axis_name="s")


@functools.partial(
    pl.kernel,
    out_type=jax.ShapeDtypeStruct((NC, NP), jnp.float32),
    mesh=_mesh(),
    scratch_types=[
        pltpu.VMEM((NCHD, CHD), jnp.int32),
        pltpu.VMEM((CHD,), jnp.float32),
        pltpu.VMEM((PT,), jnp.float32),
        pltpu.VMEM_SHARED((NP,), jnp.float32),
    ],
)
def _deg_kernel(dst_hbm, out_hbm, idx_v, ones_v, buf_v, acc_sh):
    c = lax.axis_index("c")
    s = lax.axis_index("s")
    for i in range(CHD // 16):
        ones_v[pl.ds(i * 16, 16)] = jnp.ones((16,), jnp.float32)
    for i in range(PT // 16):
        buf_v[pl.ds(i * 16, 16)] = jnp.zeros((16,), jnp.float32)
    pltpu.sync_copy(buf_v, acc_sh.at[pl.ds(s * PT, PT)])
    plsc.subcore_barrier()
    pltpu.sync_copy(dst_hbm.at[c, s], idx_v)

    def chunk(k, carry):
        pltpu.sync_copy(ones_v, acc_sh.at[idx_v.at[k]], add=True)
        return carry

    lax.fori_loop(0, NCHD, chunk, 0)
    plsc.subcore_barrier()
    pltpu.sync_copy(acc_sh.at[pl.ds(s * PT, PT)], buf_v)
    pltpu.sync_copy(buf_v, out_hbm.at[c, pl.ds(s * PT, PT)])


def _prologue_body(feat_ref, degp_ref, g0_ref, fac_ref):
    i = pl.program_id(0)
    deg = degp_ref[0, :] + degp_ref[1, :] + 1.0
    isq = lax.rsqrt(deg)
    fac_ref[0, :] = isq * isq
    fac_ref[1, :] = isq
    isq_blk = isq[pl.ds(i * 1024, 1024)]
    g0_ref[...] = feat_ref[...] * isq_blk[:, None]


def _prologue(featp, degp):
    return pl.pallas_call(
        _prologue_body,
        grid=(NP // 1024, 2),
        in_specs=[
            pl.BlockSpec((1024, D2), lambda i, c: (i, c)),
            pl.BlockSpec((NC, NP), lambda i, c: (0, 0)),
        ],
        out_specs=[
            pl.BlockSpec((1024, D2), lambda i, c: (c * (NP // 1024) + i, 0)),
            pl.BlockSpec((NC, NP), lambda i, c: (0, 0)),
        ],
        out_shape=[
            jax.ShapeDtypeStruct((2 * NP, D2), jnp.float32),
            jax.ShapeDtypeStruct((NC, NP), jnp.float32),
        ],
    )(featp, degp)


@functools.partial(
    pl.kernel,
    out_type=jax.ShapeDtypeStruct((2 * NP, D2), jnp.float32),
    mesh=_mesh(),
    scratch_types=[
        pltpu.HBM((2 * NP, D2), jnp.float32),
        pltpu.VMEM((NCH, CH), jnp.int32),
        pltpu.VMEM((NCH, CH), jnp.int32),
        pltpu.VMEM((CH, D2), jnp.float32),
        pltpu.VMEM((RC, D2), jnp.float32),
        pltpu.VMEM((RC,), jnp.float32),
        pltpu.VMEM_SHARED((NP, D2), jnp.float32),
        pltpu.SemaphoreType.DMA,
    ],
)
def _prop_kernel(g0_hbm, src_hbm, dst_hbm, fac_hbm, out_hbm, tmp_hbm,
                 src_v, dst_v, rows_v, rbuf_v, fbuf_v, acc_sh, sem):
    c = lax.axis_index("c")
    s = lax.axis_index("s")
    pltpu.sync_copy(src_hbm.at[c, s], src_v)
    pltpu.sync_copy(dst_hbm.at[s], dst_v)
    my_row0 = s * PT
    g_row0 = c * NP + my_row0

    def do_round(g_in, g_out, fac_row):
        for j in range(NRC):
            pltpu.sync_copy(g_in.at[pl.ds(g_row0 + j * RC, RC)], rbuf_v)
            pltpu.sync_copy(rbuf_v, acc_sh.at[pl.ds(my_row0 + j * RC, RC)])
        plsc.subcore_barrier()

        def chunk(k, carry):
            pltpu.async_copy(g_in.at[src_v.at[k]], rows_v, sem).wait()
            pltpu.sync_copy(rows_v, acc_sh.at[dst_v.at[k]], add=True)
            return carry

        lax.fori_loop(0, NCH, chunk, 0)
        plsc.subcore_barrier()

        for j in range(NRC):
            pltpu.sync_copy(acc_sh.at[pl.ds(my_row0 + j * RC, RC)], rbuf_v)
            pltpu.sync_copy(fac_hbm.at[fac_row, pl.ds(my_row0 + j * RC, RC)],
                            fbuf_v)

            def row(i, carry):
                f = fbuf_v[i]
                for q in range(D2 // 16):
                    rbuf_v[i, pl.ds(q * 16, 16)] = (
                        rbuf_v[i, pl.ds(q * 16, 16)] * f)
                return carry

            lax.fori_loop(0, RC, row, 0)
            pltpu.sync_copy(rbuf_v, g_out.at[pl.ds(g_row0 + j * RC, RC)])

    do_round(g0_hbm, out_hbm, 0)
    do_round(out_hbm, tmp_hbm, 0)
    do_round(tmp_hbm, out_hbm, 1)


def _matmul_body(a0_ref, a1_ref, w_ref, b_ref, out_ref):
    w0 = w_ref[:, :D2]
    w1 = w_ref[:, D2:]
    acc = lax.dot_general(a0_ref[...], w0, (((1,), (1,)), ((), ())),
                          preferred_element_type=jnp.float32)
    acc += lax.dot_general(a1_ref[...], w1, (((1,), (1,)), ((), ())),
                           preferred_element_type=jnp.float32)
    out_ref[...] = acc + b_ref[0, :][None, :]


def _matmul(hk, W, b):
    return pl.pallas_call(
        _matmul_body,
        grid=(NP // PT,),
        in_specs=[
            pl.BlockSpec((PT, D2), lambda i: (i, 0)),
            pl.BlockSpec((PT, D2), lambda i: (NP // PT + i, 0)),
            pl.BlockSpec((NCLS, D), lambda i: (0, 0)),
            pl.BlockSpec((1, NCLS), lambda i: (0, 0)),
        ],
        out_specs=pl.BlockSpec((PT, NCLS), lambda i: (i, 0)),
        out_shape=jax.ShapeDtypeStruct((NP, NCLS), jnp.float32),
    )(hk, hk, W, b.reshape(1, NCLS))


def kernel(feat, edge_index, W, b):
    src = edge_index[0].astype(jnp.int32)
    dst = edge_index[1].astype(jnp.int32)
    src_h = jnp.stack([src, src + NP]).reshape(NC, NS, NCH, CH)
    dst_h = dst.reshape(NS, NCH, CH)
    dst_d = dst.reshape(NC, NS, NCHD, CHD)
    featp = jnp.pad(feat, ((0, NP - N), (0, 0)))

    degp = _deg_kernel(dst_d)
    g0, fac = _prologue(featp, degp)
    hk = _prop_kernel(g0, src_h, dst_h, fac)
    out = _matmul(hk, W, b)
    return out[:N]


# speedup vs baseline: 4.1459x; 4.1459x over previous
import functools

import jax
import jax.numpy as jnp
from jax import lax
from jax.experimental import pallas as pl
from jax.experimental.pallas import tpu as pltpu
from jax.experimental.pallas import tpu_sc as plsc

N = 10000
NP = 10240
E = 160000
D = 256
D2 = 128
NCLS = 128
K = 3

NC = 2
NS = 16

E2 = 163840
ET = E2 // NS
CH = 80
NCH = ET // CH
PT = NP // NS
RC = 64
NRC = PT // RC
WN = 16
NW = NCH // WN

ED = E // (NC * NS)
CHD = 40
NCHD = ED // CHD

_mesh = functools.partial(
    plsc.VectorSubcoreMesh, core_axis_name="c", subcore_axis_name="s")


@functools.partial(
    pl.kernel,
    out_type=jax.ShapeDtypeStruct((NC, NP), jnp.float32),
    mesh=_mesh(),
    scratch_types=[
        pltpu.VMEM((NCHD, CHD), jnp.int32),
        pltpu.VMEM((CHD,), jnp.float32),
        pltpu.VMEM((PT,), jnp.float32),
        pltpu.VMEM_SHARED((NP,), jnp.float32),
    ],
)
def _deg_kernel(dst_hbm, out_hbm, idx_v, ones_v, buf_v, acc_sh):
    c = lax.axis_index("c")
    s = lax.axis_index("s")
    for i in range(CHD // 16):
        ones_v[pl.ds(i * 16, 16)] = jnp.ones((16,), jnp.float32)
    for i in range(PT // 16):
        buf_v[pl.ds(i * 16, 16)] = jnp.zeros((16,), jnp.float32)
    pltpu.sync_copy(buf_v, acc_sh.at[pl.ds(s * PT, PT)])
    plsc.subcore_barrier()
    pltpu.sync_copy(dst_hbm.at[c, s], idx_v)

    def chunk(k, carry):
        pltpu.sync_copy(ones_v, acc_sh.at[idx_v.at[k]], add=True)
        return carry

    lax.fori_loop(0, NCHD, chunk, 0)
    plsc.subcore_barrier()
    pltpu.sync_copy(acc_sh.at[pl.ds(s * PT, PT)], buf_v)
    pltpu.sync_copy(buf_v, out_hbm.at[c, pl.ds(s * PT, PT)])


def _prologue_body(feat_ref, degp_ref, g0_ref, fac_ref):
    i = pl.program_id(0)
    deg = degp_ref[0, :] + degp_ref[1, :] + 1.0
    isq = lax.rsqrt(deg)
    fac_ref[0, :] = isq * isq
    fac_ref[1, :] = isq
    d0 = degp_ref[0, pl.ds(i * 1024, 1024)]
    d1 = degp_ref[1, pl.ds(i * 1024, 1024)]
    isq_blk = lax.rsqrt(d0 + d1 + 1.0)
    g0_ref[...] = feat_ref[...] * isq_blk[:, None]


def _prologue(featp, degp):
    return pl.pallas_call(
        _prologue_body,
        grid=(NP // 1024, 2),
        in_specs=[
            pl.BlockSpec((1024, D2), lambda i, c: (i, c)),
            pl.BlockSpec((NC, NP), lambda i, c: (0, 0)),
        ],
        out_specs=[
            pl.BlockSpec((1024, D2), lambda i, c: (c * (NP // 1024) + i, 0)),
            pl.BlockSpec((NC, NP), lambda i, c: (0, 0)),
        ],
        out_shape=[
            jax.ShapeDtypeStruct((2 * NP, D2), jnp.float32),
            jax.ShapeDtypeStruct((NC, NP), jnp.float32),
        ],
    )(featp, degp)


@functools.partial(
    pl.kernel,
    out_type=jax.ShapeDtypeStruct((2 * NP, D2), jnp.float32),
    mesh=_mesh(),
    scratch_types=[
        pltpu.HBM((2 * NP, D2), jnp.float32),
        pltpu.VMEM((WN, CH), jnp.int32),
        pltpu.VMEM((WN, CH), jnp.int32),
        pltpu.VMEM((CH, D2), jnp.float32),
        pltpu.VMEM((RC, D2), jnp.float32),
        pltpu.VMEM((PT,), jnp.float32),
        pltpu.VMEM_SHARED((NP, D2), jnp.float32),
        pltpu.SemaphoreType.DMA,
    ],
)
def _prop_kernel(g0_hbm, src_hbm, dst_hbm, fac_hbm, out_hbm, tmp_hbm,
                 src_v, dst_v, rows_v, rbuf_v, fbuf_v, acc_sh, sem):
    c = lax.axis_index("c")
    s = lax.axis_index("s")
    my_row0 = s * PT
    g_row0 = c * NP + my_row0

    def do_round(g_in, g_out, fac_row):
        def seed(j, carry):
            pltpu.sync_copy(g_in.at[pl.ds(g_row0 + j * RC, RC)], rbuf_v)
            pltpu.sync_copy(rbuf_v, acc_sh.at[pl.ds(my_row0 + j * RC, RC)])
            return carry

        lax.fori_loop(0, NRC, seed, 0)
        plsc.subcore_barrier()

        def window(w, carry):
            pltpu.sync_copy(src_hbm.at[c, s, pl.ds(w * WN, WN)], src_v)
            pltpu.sync_copy(dst_hbm.at[s, pl.ds(w * WN, WN)], dst_v)

            def chunk(k, carry2):
                pltpu.async_copy(g_in.at[src_v.at[k]], rows_v, sem).wait()
                pltpu.sync_copy(rows_v, acc_sh.at[dst_v.at[k]], add=True)
                return carry2

            lax.fori_loop(0, WN, chunk, 0)
            return carry

        lax.fori_loop(0, NW, window, 0)
        plsc.subcore_barrier()

        pltpu.sync_copy(fac_hbm.at[fac_row, pl.ds(my_row0, PT)], fbuf_v)

        def scale(j, carry):
            pltpu.sync_copy(acc_sh.at[pl.ds(my_row0 + j * RC, RC)], rbuf_v)

            def row16(t, carry2):
                fvec = fbuf_v[pl.ds(j * RC + t * 16, 16)]
                for l in range(16):
                    f = fvec[l]
                    i = t * 16 + l
                    for q in range(D2 // 16):
                        rbuf_v[i, pl.ds(q * 16, 16)] = (
                            rbuf_v[i, pl.ds(q * 16, 16)] * f)
                return carry2

            lax.fori_loop(0, RC // 16, row16, 0)
            pltpu.sync_copy(rbuf_v, g_out.at[pl.ds(g_row0 + j * RC, RC)])
            return carry

        lax.fori_loop(0, NRC, scale, 0)

    do_round(g0_hbm, out_hbm, 0)
    do_round(out_hbm, tmp_hbm, 0)
    do_round(tmp_hbm, out_hbm, 1)


def _matmul_body(a0_ref, a1_ref, w_ref, b_ref, out_ref):
    w0 = w_ref[:, :D2]
    w1 = w_ref[:, D2:]
    acc = lax.dot_general(a0_ref[...], w0, (((1,), (1,)), ((), ())),
                          preferred_element_type=jnp.float32)
    acc += lax.dot_general(a1_ref[...], w1, (((1,), (1,)), ((), ())),
                           preferred_element_type=jnp.float32)
    out_ref[...] = acc + b_ref[0, :][None, :]


def _matmul(hk, W, b):
    return pl.pallas_call(
        _matmul_body,
        grid=(NP // PT,),
        in_specs=[
            pl.BlockSpec((PT, D2), lambda i: (i, 0)),
            pl.BlockSpec((PT, D2), lambda i: (NP // PT + i, 0)),
            pl.BlockSpec((NCLS, D), lambda i: (0, 0)),
            pl.BlockSpec((1, NCLS), lambda i: (0, 0)),
        ],
        out_specs=pl.BlockSpec((PT, NCLS), lambda i: (i, 0)),
        out_shape=jax.ShapeDtypeStruct((NP, NCLS), jnp.float32),
    )(hk, hk, W, b.reshape(1, NCLS))


def kernel(feat, edge_index, W, b):
    src = edge_index[0].astype(jnp.int32)
    dst = edge_index[1].astype(jnp.int32)
    pad = jnp.full((E2 - E,), NP - 1, jnp.int32)
    src_p = jnp.concatenate([src, pad])
    dst_p = jnp.concatenate([dst, pad])
    src_h = jnp.stack([src_p, src_p + NP]).reshape(NC, NS, NCH, CH)
    dst_h = dst_p.reshape(NS, NCH, CH)
    dst_d = dst.reshape(NC, NS, NCHD, CHD)
    featp = jnp.pad(feat, ((0, NP - N), (0, 0)))

    degp = _deg_kernel(dst_d)
    g0, fac = _prologue(featp, degp)
    hk = _prop_kernel(g0, src_h, dst_h, fac)
    out = _matmul(hk, W, b)
    return out[:N]


# speedup vs baseline: 4.6476x; 1.1210x over previous
import functools

import jax
import jax.numpy as jnp
from jax import lax
from jax.experimental import pallas as pl
from jax.experimental.pallas import tpu as pltpu
from jax.experimental.pallas import tpu_sc as plsc

N = 10000
NP = 10240
E = 160000
D = 256
D2 = 128
NCLS = 128
K = 3

NC = 2
NS = 16

E2 = 163840
ET = E2 // NS
CH = 80
NCH = ET // CH
PT = NP // NS
RC = 64
NRC = PT // RC
WN = 16
NW = NCH // WN

ED = E // (NC * NS)
CHD = 40
NCHD = ED // CHD

_mesh = functools.partial(
    plsc.VectorSubcoreMesh, core_axis_name="c", subcore_axis_name="s")


@functools.partial(
    pl.kernel,
    out_type=jax.ShapeDtypeStruct((NC, NP), jnp.float32),
    mesh=_mesh(),
    scratch_types=[
        pltpu.VMEM((NCHD, CHD), jnp.int32),
        pltpu.VMEM((CHD,), jnp.float32),
        pltpu.VMEM((PT,), jnp.float32),
        pltpu.VMEM_SHARED((NP,), jnp.float32),
    ],
)
def _deg_kernel(dst_hbm, out_hbm, idx_v, ones_v, buf_v, acc_sh):
    c = lax.axis_index("c")
    s = lax.axis_index("s")
    for i in range(CHD // 16):
        ones_v[pl.ds(i * 16, 16)] = jnp.ones((16,), jnp.float32)
    for i in range(PT // 16):
        buf_v[pl.ds(i * 16, 16)] = jnp.zeros((16,), jnp.float32)
    pltpu.sync_copy(buf_v, acc_sh.at[pl.ds(s * PT, PT)])
    plsc.subcore_barrier()
    pltpu.sync_copy(dst_hbm.at[c, s], idx_v)

    def chunk(k, carry):
        pltpu.sync_copy(ones_v, acc_sh.at[idx_v.at[k]], add=True)
        return carry

    lax.fori_loop(0, NCHD, chunk, 0)
    plsc.subcore_barrier()
    pltpu.sync_copy(acc_sh.at[pl.ds(s * PT, PT)], buf_v)
    pltpu.sync_copy(buf_v, out_hbm.at[c, pl.ds(s * PT, PT)])


def _prologue_body(feat_ref, degp_ref, g0_ref, fac_ref):
    i = pl.program_id(0)
    deg = degp_ref[0, :] + degp_ref[1, :] + 1.0
    isq = lax.rsqrt(deg)
    fac_ref[0, :] = isq * isq
    fac_ref[1, :] = isq
    d0 = degp_ref[0, pl.ds(i * 1024, 1024)]
    d1 = degp_ref[1, pl.ds(i * 1024, 1024)]
    isq_blk = lax.rsqrt(d0 + d1 + 1.0)
    g0_ref[...] = feat_ref[...] * isq_blk[:, None]


def _prologue(featp, degp):
    return pl.pallas_call(
        _prologue_body,
        grid=(NP // 1024, 2),
        in_specs=[
            pl.BlockSpec((1024, D2), lambda i, c: (i, c)),
            pl.BlockSpec((NC, NP), lambda i, c: (0, 0)),
        ],
        out_specs=[
            pl.BlockSpec((1024, D2), lambda i, c: (c * (NP // 1024) + i, 0)),
            pl.BlockSpec((NC, NP), lambda i, c: (0, 0)),
        ],
        out_shape=[
            jax.ShapeDtypeStruct((2 * NP, D2), jnp.float32),
            jax.ShapeDtypeStruct((NC, NP), jnp.float32),
        ],
    )(featp, degp)


@functools.partial(
    pl.kernel,
    out_type=jax.ShapeDtypeStruct((2 * NP, D2), jnp.float32),
    mesh=_mesh(),
    scratch_types=[
        pltpu.HBM((2 * NP, D2), jnp.float32),
        pltpu.VMEM((WN, CH), jnp.int32),
        pltpu.VMEM((WN, CH), jnp.int32),
        pltpu.VMEM((CH, D2), jnp.float32),
        pltpu.VMEM((CH, D2), jnp.float32),
        pltpu.VMEM((RC, D2), jnp.float32),
        pltpu.VMEM((PT,), jnp.float32),
        pltpu.VMEM_SHARED((NP, D2), jnp.float32),
        pltpu.SemaphoreType.DMA,
        pltpu.SemaphoreType.DMA,
        pltpu.SemaphoreType.DMA,
        pltpu.SemaphoreType.DMA,
    ],
)
def _prop_kernel(g0_hbm, src_hbm, dst_hbm, fac_hbm, out_hbm, tmp_hbm,
                 src_v, dst_v, rows0_v, rows1_v, rbuf_v, fbuf_v, acc_sh,
                 gsem0, gsem1, ssem0, ssem1):
    c = lax.axis_index("c")
    s = lax.axis_index("s")
    my_row0 = s * PT
    g_row0 = c * NP + my_row0

    def do_round(g_in, g_out, fac_row):
        def seed(j, carry):
            pltpu.sync_copy(g_in.at[pl.ds(g_row0 + j * RC, RC)], rbuf_v)
            pltpu.sync_copy(rbuf_v, acc_sh.at[pl.ds(my_row0 + j * RC, RC)])
            return carry

        lax.fori_loop(0, NRC, seed, 0)
        plsc.subcore_barrier()

        gsems = (gsem0, gsem1)
        rows = (rows0_v, rows1_v)

        def window(w, carry):
            pltpu.sync_copy(src_hbm.at[c, s, pl.ds(w * WN, WN)], src_v)
            pltpu.sync_copy(dst_hbm.at[s, pl.ds(w * WN, WN)], dst_v)
            gath = [None] * WN
            gath[0] = pltpu.async_copy(
                g_in.at[src_v.at[0]], rows[0], gsems[0])
            for k in range(WN):
                gath[k].wait()
                if k + 1 < WN:
                    gath[k + 1] = pltpu.async_copy(
                        g_in.at[src_v.at[k + 1]], rows[(k + 1) % 2],
                        gsems[(k + 1) % 2])
                pltpu.sync_copy(rows[k % 2], acc_sh.at[dst_v.at[k]],
                                add=True)
            return carry

        lax.fori_loop(0, NW, window, 0)
        plsc.subcore_barrier()

        pltpu.sync_copy(fac_hbm.at[fac_row, pl.ds(my_row0, PT)], fbuf_v)

        def scale(j, carry):
            pltpu.sync_copy(acc_sh.at[pl.ds(my_row0 + j * RC, RC)], rbuf_v)

            def row16(t, carry2):
                fvec = fbuf_v[pl.ds(j * RC + t * 16, 16)]
                for l in range(16):
                    f = fvec[l]
                    i = t * 16 + l
                    for q in range(D2 // 16):
                        rbuf_v[i, pl.ds(q * 16, 16)] = (
                            rbuf_v[i, pl.ds(q * 16, 16)] * f)
                return carry2

            lax.fori_loop(0, RC // 16, row16, 0)
            pltpu.sync_copy(rbuf_v, g_out.at[pl.ds(g_row0 + j * RC, RC)])
            return carry

        lax.fori_loop(0, NRC, scale, 0)

    do_round(g0_hbm, out_hbm, 0)
    do_round(out_hbm, tmp_hbm, 0)
    do_round(tmp_hbm, out_hbm, 1)


def _matmul_body(a0_ref, a1_ref, w_ref, b_ref, out_ref):
    w0 = w_ref[:, :D2]
    w1 = w_ref[:, D2:]
    acc = lax.dot_general(a0_ref[...], w0, (((1,), (1,)), ((), ())),
                          preferred_element_type=jnp.float32)
    acc += lax.dot_general(a1_ref[...], w1, (((1,), (1,)), ((), ())),
                           preferred_element_type=jnp.float32)
    out_ref[...] = acc + b_ref[0, :][None, :]


def _matmul(hk, W, b):
    return pl.pallas_call(
        _matmul_body,
        grid=(NP // PT,),
        in_specs=[
            pl.BlockSpec((PT, D2), lambda i: (i, 0)),
            pl.BlockSpec((PT, D2), lambda i: (NP // PT + i, 0)),
            pl.BlockSpec((NCLS, D), lambda i: (0, 0)),
            pl.BlockSpec((1, NCLS), lambda i: (0, 0)),
        ],
        out_specs=pl.BlockSpec((PT, NCLS), lambda i: (i, 0)),
        out_shape=jax.ShapeDtypeStruct((NP, NCLS), jnp.float32),
    )(hk, hk, W, b.reshape(1, NCLS))


def kernel(feat, edge_index, W, b):
    src = edge_index[0].astype(jnp.int32)
    dst = edge_index[1].astype(jnp.int32)
    pad = jnp.full((E2 - E,), NP - 1, jnp.int32)
    src_p = jnp.concatenate([src, pad])
    dst_p = jnp.concatenate([dst, pad])
    src_h = jnp.stack([src_p, src_p + NP]).reshape(NC, NS, NCH, CH)
    dst_h = dst_p.reshape(NS, NCH, CH)
    dst_d = dst.reshape(NC, NS, NCHD, CHD)
    featp = jnp.pad(feat, ((0, NP - N), (0, 0)))

    degp = _deg_kernel(dst_d)
    g0, fac = _prologue(featp, degp)
    hk = _prop_kernel(g0, src_h, dst_h, fac)
    out = _matmul(hk, W, b)
    return out[:N]
